# Initial kernel scaffold; baseline (speedup 1.0000x reference)
import jax
import jax.numpy as jnp
from jax.experimental import pallas as pl


def kernel(ae_q, ae_kv, pe_q, pe_kv, Wq, Wk):
    raise NotImplementedError("write your pallas kernel here")


# speedup vs baseline: 8.3917x; 8.3917x over previous
import math
import functools

import jax
import jax.numpy as jnp
from jax.experimental import pallas as pl
from jax.experimental.pallas import tpu as pltpu

B = 4096
KNOW = 4096
D = 1024
TOP_K = 64
SCALE = 1.0 / 32.0


def _proj_kernel(a_ref, p_ref, w_ref, o_ref):
    r = (a_ref[...] + p_ref[...]) * 0.5
    o_ref[...] = (
        jax.lax.dot_general(
            r, w_ref[...], (((1,), (1,)), ((), ())),
            preferred_element_type=jnp.float32,
        )
        + r
    )


def _sims_kernel(pq_ref, pk_ref, o_ref):
    o_ref[...] = (
        jax.lax.dot_general(
            pq_ref[...], pk_ref[...], (((1,), (1,)), ((), ())),
            preferred_element_type=jnp.float32,
        )
        * SCALE
    )


def _kth_kernel(s_ref, o_ref):
    s = s_ref[...] + 0.0
    i = jax.lax.bitcast_convert_type(s, jnp.int32)
    key = i ^ (jnp.right_shift(i, 31) & jnp.int32(0x7FFFFFFF))

    def cnt_ge(c):
        return jnp.sum((key >= c).astype(jnp.int32), axis=1, keepdims=True)

    rows = s.shape[0]
    int_min = jnp.int32(-(2**31))
    zero = jnp.zeros((rows, 1), jnp.int32)
    res = jnp.where(cnt_ge(zero) >= TOP_K, zero, jnp.full((rows, 1), int_min))
    for b in range(30, -1, -1):
        cand = res | jnp.int32(1 << b)
        res = jnp.where(cnt_ge(cand) >= TOP_K, cand, res)
    kth_i = res ^ (jnp.right_shift(res, 31) & jnp.int32(0x7FFFFFFF))
    o_ref[...] = jax.lax.bitcast_convert_type(kth_i, jnp.float32)


def _att_kernel(q_ref, k_ref, s_ref, t_ref, o_ref):
    att = (
        jax.lax.dot_general(
            q_ref[...], k_ref[...], (((1,), (1,)), ((), ())),
            preferred_element_type=jnp.float32,
        )
        * SCALE
    )
    o_ref[...] = jnp.where(s_ref[...] < t_ref[...], jnp.float32(0.0), att)


@jax.jit
def kernel(ae_q, ae_kv, pe_q, pe_kv, Wq, Wk):
    proj = pl.pallas_call(
        _proj_kernel,
        grid=(8,),
        in_specs=[
            pl.BlockSpec((512, D), lambda i: (i, 0)),
            pl.BlockSpec((512, D), lambda i: (i, 0)),
            pl.BlockSpec((D, D), lambda i: (0, 0)),
        ],
        out_specs=pl.BlockSpec((512, D), lambda i: (i, 0)),
        out_shape=jax.ShapeDtypeStruct((B, D), jnp.float32),
    )
    query = proj(ae_q, pe_q, Wq)
    key_mat = proj(ae_kv, pe_kv, Wk)

    sims = pl.pallas_call(
        _sims_kernel,
        grid=(2, 8),
        in_specs=[
            pl.BlockSpec((512, D), lambda j, i: (i, 0)),
            pl.BlockSpec((2048, D), lambda j, i: (j, 0)),
        ],
        out_specs=pl.BlockSpec((512, 2048), lambda j, i: (i, j)),
        out_shape=jax.ShapeDtypeStruct((B, KNOW), jnp.float32),
    )(pe_q, pe_kv)

    kth = pl.pallas_call(
        _kth_kernel,
        grid=(16,),
        in_specs=[pl.BlockSpec((256, KNOW), lambda i: (i, 0))],
        out_specs=pl.BlockSpec((256, 1), lambda i: (i, 0)),
        out_shape=jax.ShapeDtypeStruct((B, 1), jnp.float32),
    )(sims)

    out = pl.pallas_call(
        _att_kernel,
        grid=(4, 16),
        in_specs=[
            pl.BlockSpec((256, D), lambda j, i: (i, 0)),
            pl.BlockSpec((1024, D), lambda j, i: (j, 0)),
            pl.BlockSpec((256, 1024), lambda j, i: (i, j)),
            pl.BlockSpec((256, 1), lambda j, i: (i, 0)),
        ],
        out_specs=pl.BlockSpec((256, 1024), lambda j, i: (i, j)),
        out_shape=jax.ShapeDtypeStruct((B, KNOW), jnp.float32),
    )(query, key_mat, sims, kth)
    return out


# speedup vs baseline: 8.7866x; 1.0471x over previous
import math
import functools

import jax
import jax.numpy as jnp
from jax.experimental import pallas as pl
from jax.experimental.pallas import tpu as pltpu

B = 4096
KNOW = 4096
D = 1024
TOP_K = 64
SCALE = 1.0 / 32.0


def _proj_kernel(a_ref, p_ref, w_ref, o_ref):
    r = (a_ref[...] + p_ref[...]) * 0.5
    o_ref[...] = (
        jax.lax.dot_general(
            r, w_ref[...], (((1,), (1,)), ((), ())),
            preferred_element_type=jnp.float32,
        )
        + r
    )


def _row_kth(s):
    i = jax.lax.bitcast_convert_type(s + 0.0, jnp.int32)
    key = i ^ (jnp.right_shift(i, 31) & jnp.int32(0x7FFFFFFF))

    def cnt_ge(c):
        return jnp.sum((key >= c).astype(jnp.int32), axis=1, keepdims=True)

    rows = s.shape[0]
    int_min = jnp.int32(-(2**31))
    zero = jnp.zeros((rows, 1), jnp.int32)
    res = jnp.where(cnt_ge(zero) >= TOP_K, zero, jnp.full((rows, 1), int_min))
    for b in range(30, -1, -1):
        cand = res | jnp.int32(1 << b)
        res = jnp.where(cnt_ge(cand) >= TOP_K, cand, res)
    kth_i = res ^ (jnp.right_shift(res, 31) & jnp.int32(0x7FFFFFFF))
    return jax.lax.bitcast_convert_type(kth_i, jnp.float32)


def _fused_kernel(pq_ref, pkv_ref, q_ref, k_ref, o_ref):
    sims = (
        jax.lax.dot_general(
            pq_ref[...], pkv_ref[...], (((1,), (1,)), ((), ())),
            preferred_element_type=jnp.float32,
        )
        * SCALE
    )
    kth = _row_kth(sims)
    att = (
        jax.lax.dot_general(
            q_ref[...], k_ref[...], (((1,), (1,)), ((), ())),
            preferred_element_type=jnp.float32,
        )
        * SCALE
    )
    o_ref[...] = jnp.where(sims < kth, jnp.float32(0.0), att)


@jax.jit
def kernel(ae_q, ae_kv, pe_q, pe_kv, Wq, Wk):
    proj = pl.pallas_call(
        _proj_kernel,
        grid=(8,),
        in_specs=[
            pl.BlockSpec((512, D), lambda i: (i, 0)),
            pl.BlockSpec((512, D), lambda i: (i, 0)),
            pl.BlockSpec((D, D), lambda i: (0, 0)),
        ],
        out_specs=pl.BlockSpec((512, D), lambda i: (i, 0)),
        out_shape=jax.ShapeDtypeStruct((B, D), jnp.float32),
    )
    query = proj(ae_q, pe_q, Wq)
    key_mat = proj(ae_kv, pe_kv, Wk)

    R = 128
    out = pl.pallas_call(
        _fused_kernel,
        grid=(B // R,),
        in_specs=[
            pl.BlockSpec((R, D), lambda i: (i, 0)),
            pl.BlockSpec((KNOW, D), lambda i: (0, 0)),
            pl.BlockSpec((R, D), lambda i: (i, 0)),
            pl.BlockSpec((KNOW, D), lambda i: (0, 0)),
        ],
        out_specs=pl.BlockSpec((R, KNOW), lambda i: (i, 0)),
        out_shape=jax.ShapeDtypeStruct((B, KNOW), jnp.float32),
    )(pe_q, pe_kv, query, key_mat)
    return out


# speedup vs baseline: 10.6192x; 1.2086x over previous
import math
import functools

import jax
import jax.numpy as jnp
from jax.experimental import pallas as pl
from jax.experimental.pallas import tpu as pltpu

B = 4096
KNOW = 4096
D = 1024
TOP_K = 64
SCALE = 1.0 / 32.0


def _proj_kernel(a_ref, p_ref, w_ref, o_ref):
    r = (a_ref[...] + p_ref[...]) * 0.5
    o_ref[...] = (
        jax.lax.dot_general(
            r, w_ref[...], (((1,), (1,)), ((), ())),
            preferred_element_type=jnp.float32,
        )
        + r
    )


def _row_kth(s):
    i = jax.lax.bitcast_convert_type(s + 0.0, jnp.int32)
    key = i ^ (jnp.right_shift(i, 31) & jnp.int32(0x7FFFFFFF))
    hi = jnp.right_shift(key, 16).astype(jnp.int16)
    lo = ((key ^ jnp.int32(0x8000)) & jnp.int32(0xFFFF)).astype(jnp.int16)
    one16 = jnp.int16(1)
    zero16 = jnp.int16(0)

    def fold_sum(m):
        w = m.shape[1]
        while w > 128:
            w //= 2
            m = m[:, :w] + m[:, w:]
        return jnp.sum(m.astype(jnp.int32), axis=1, keepdims=True)

    def cnt_ge(arr, c_i32):
        cb = jnp.broadcast_to(c_i32.astype(jnp.int16), arr.shape)
        return fold_sum(jnp.where(arr >= cb, one16, zero16))

    rows = s.shape[0]
    i16_min = jnp.full((rows, 1), -(2**15), jnp.int32)
    zero = jnp.zeros((rows, 1), jnp.int32)
    res = jnp.where(cnt_ge(hi, zero) >= TOP_K, zero, i16_min)
    for b in range(14, -1, -1):
        cand = res | jnp.int32(1 << b)
        res = jnp.where(cnt_ge(hi, cand) >= TOP_K, cand, res)
    p = res
    pb = jnp.broadcast_to(p.astype(jnp.int16), hi.shape)
    cnt_above = fold_sum(jnp.where(hi > pb, one16, zero16))
    target = TOP_K - cnt_above
    lob = jnp.where(hi == pb, lo, jnp.int16(-(2**15)))
    resb = jnp.where(cnt_ge(lob, zero) >= target, zero, i16_min)
    for b in range(14, -1, -1):
        cand = resb | jnp.int32(1 << b)
        resb = jnp.where(cnt_ge(lob, cand) >= target, cand, resb)
    kth_key = jnp.left_shift(p, 16) | ((resb & 0xFFFF) ^ 0x8000)
    kth_i = kth_key ^ (jnp.right_shift(kth_key, 31) & jnp.int32(0x7FFFFFFF))
    return jax.lax.bitcast_convert_type(kth_i, jnp.float32)


def _fused_kernel(pq_ref, pkv_ref, q_ref, k_ref, o_ref):
    sims = (
        jax.lax.dot_general(
            pq_ref[...], pkv_ref[...], (((1,), (1,)), ((), ())),
            preferred_element_type=jnp.float32,
        )
        * SCALE
    )
    kth = _row_kth(sims)
    att = (
        jax.lax.dot_general(
            q_ref[...], k_ref[...], (((1,), (1,)), ((), ())),
            preferred_element_type=jnp.float32,
        )
        * SCALE
    )
    o_ref[...] = jnp.where(sims < kth, jnp.float32(0.0), att)


@jax.jit
def kernel(ae_q, ae_kv, pe_q, pe_kv, Wq, Wk):
    proj = pl.pallas_call(
        _proj_kernel,
        grid=(8,),
        in_specs=[
            pl.BlockSpec((512, D), lambda i: (i, 0)),
            pl.BlockSpec((512, D), lambda i: (i, 0)),
            pl.BlockSpec((D, D), lambda i: (0, 0)),
        ],
        out_specs=pl.BlockSpec((512, D), lambda i: (i, 0)),
        out_shape=jax.ShapeDtypeStruct((B, D), jnp.float32),
    )
    query = proj(ae_q, pe_q, Wq)
    key_mat = proj(ae_kv, pe_kv, Wk)

    R = 128
    out = pl.pallas_call(
        _fused_kernel,
        grid=(B // R,),
        in_specs=[
            pl.BlockSpec((R, D), lambda i: (i, 0)),
            pl.BlockSpec((KNOW, D), lambda i: (0, 0)),
            pl.BlockSpec((R, D), lambda i: (i, 0)),
            pl.BlockSpec((KNOW, D), lambda i: (0, 0)),
        ],
        out_specs=pl.BlockSpec((R, KNOW), lambda i: (i, 0)),
        out_shape=jax.ShapeDtypeStruct((B, KNOW), jnp.float32),
    )(pe_q, pe_kv, query, key_mat)
    return out


# speedup vs baseline: 13.1762x; 1.2408x over previous
import math
import functools

import jax
import jax.numpy as jnp
from jax.experimental import pallas as pl
from jax.experimental.pallas import tpu as pltpu

B = 4096
KNOW = 4096
D = 1024
TOP_K = 64
SCALE = 1.0 / 32.0


def _proj_kernel(a_ref, p_ref, w_ref, o_ref):
    r = (a_ref[...] + p_ref[...]) * 0.5
    o_ref[...] = (
        jax.lax.dot_general(
            r, w_ref[...], (((1,), (1,)), ((), ())),
            preferred_element_type=jnp.float32,
        )
        + r
    )


def _row_kth(s):
    i = jax.lax.bitcast_convert_type(s + 0.0, jnp.int32)
    key = i ^ (jnp.right_shift(i, 31) & jnp.int32(0x7FFFFFFF))
    hi = jnp.right_shift(key, 16).astype(jnp.int16)
    lo = ((key ^ jnp.int32(0x8000)) & jnp.int32(0xFFFF)).astype(jnp.int16)
    one16 = jnp.int16(1)
    zero16 = jnp.int16(0)

    def fold_sum(m):
        w = m.shape[1]
        while w > 128:
            w //= 2
            m = m[:, :w] + m[:, w:]
        return jnp.sum(m.astype(jnp.int32), axis=1, keepdims=True)

    def cnt_ge(arr, c_i32):
        cb = jnp.broadcast_to(c_i32.astype(jnp.int16), arr.shape)
        return fold_sum(jnp.where(arr >= cb, one16, zero16))

    rows = s.shape[0]
    i16_min = jnp.full((rows, 1), -(2**15), jnp.int32)
    zero = jnp.zeros((rows, 1), jnp.int32)
    res = jnp.where(cnt_ge(hi, zero) >= TOP_K, zero, i16_min)
    for b in range(14, -1, -1):
        cand = res | jnp.int32(1 << b)
        res = jnp.where(cnt_ge(hi, cand) >= TOP_K, cand, res)
    p = res
    pb = jnp.broadcast_to(p.astype(jnp.int16), hi.shape)
    cnt_above = fold_sum(jnp.where(hi > pb, one16, zero16))
    target = TOP_K - cnt_above
    lob = jnp.where(hi == pb, lo, jnp.int16(-(2**15)))
    resb = jnp.where(cnt_ge(lob, zero) >= target, zero, i16_min)
    for b in range(14, -1, -1):
        cand = resb | jnp.int32(1 << b)
        resb = jnp.where(cnt_ge(lob, cand) >= target, cand, resb)
    kth_key = jnp.left_shift(p, 16) | ((resb & 0xFFFF) ^ 0x8000)
    kth_i = kth_key ^ (jnp.right_shift(kth_key, 31) & jnp.int32(0x7FFFFFFF))
    return jax.lax.bitcast_convert_type(kth_i, jnp.float32)


def _fused_kernel(pq_ref, pkv_ref, q_ref, k_ref, o_ref):
    sims = (
        jax.lax.dot_general(
            pq_ref[...], pkv_ref[...], (((1,), (1,)), ((), ())),
            preferred_element_type=jnp.float32,
        )
        * SCALE
    )
    kth = _row_kth(sims)
    att = (
        jax.lax.dot_general(
            q_ref[...], k_ref[...], (((1,), (1,)), ((), ())),
            preferred_element_type=jnp.float32,
        )
        * SCALE
    )
    o_ref[...] = jnp.where(sims < kth, jnp.float32(0.0), att)


@jax.jit
def kernel(ae_q, ae_kv, pe_q, pe_kv, Wq, Wk):
    proj = pl.pallas_call(
        _proj_kernel,
        grid=(8,),
        in_specs=[
            pl.BlockSpec((512, D), lambda i: (i, 0)),
            pl.BlockSpec((512, D), lambda i: (i, 0)),
            pl.BlockSpec((D, D), lambda i: (0, 0)),
        ],
        out_specs=pl.BlockSpec((512, D), lambda i: (i, 0)),
        out_shape=jax.ShapeDtypeStruct((B, D), jnp.float32),
    )
    query = proj(ae_q, pe_q, Wq)
    key_mat = proj(ae_kv, pe_kv, Wk)

    R = 256
    out = pl.pallas_call(
        _fused_kernel,
        grid=(B // R,),
        in_specs=[
            pl.BlockSpec((R, D), lambda i: (i, 0)),
            pl.BlockSpec((KNOW, D), lambda i: (0, 0)),
            pl.BlockSpec((R, D), lambda i: (i, 0)),
            pl.BlockSpec((KNOW, D), lambda i: (0, 0)),
        ],
        out_specs=pl.BlockSpec((R, KNOW), lambda i: (i, 0)),
        out_shape=jax.ShapeDtypeStruct((B, KNOW), jnp.float32),
    )(pe_q, pe_kv, query, key_mat)
    return out


# speedup vs baseline: 13.7656x; 1.0447x over previous
import math
import functools

import jax
import jax.numpy as jnp
from jax.experimental import pallas as pl
from jax.experimental.pallas import tpu as pltpu

B = 4096
KNOW = 4096
D = 1024
TOP_K = 64
SCALE = 1.0 / 32.0


def _proj_kernel(a_ref, p_ref, w_ref, o_ref):
    r = (a_ref[...] + p_ref[...]) * 0.5
    o_ref[...] = (
        jax.lax.dot_general(
            r, w_ref[...], (((1,), (1,)), ((), ())),
            preferred_element_type=jnp.float32,
        )
        + r
    )


def _row_kth(s):
    i = jax.lax.bitcast_convert_type(s + 0.0, jnp.int32)
    key = i ^ (jnp.right_shift(i, 31) & jnp.int32(0x7FFFFFFF))
    hi = jnp.right_shift(key, 16).astype(jnp.int16)
    lo = ((key ^ jnp.int32(0x8000)) & jnp.int32(0xFFFF)).astype(jnp.int16)
    one16 = jnp.int16(1)
    zero16 = jnp.int16(0)

    def fold_sum(m):
        w = m.shape[1]
        while w > 128:
            w //= 2
            m = m[:, :w] + m[:, w:]
        return jnp.sum(m.astype(jnp.float32), axis=1, keepdims=True).astype(jnp.int32)

    def cnt_ge(arr, c_i32):
        cb = jnp.broadcast_to(c_i32.astype(jnp.int16), arr.shape)
        return fold_sum(jnp.where(arr >= cb, one16, zero16))

    rows = s.shape[0]
    i16_min = jnp.full((rows, 1), -(2**15), jnp.int32)
    zero = jnp.zeros((rows, 1), jnp.int32)
    res = jnp.where(cnt_ge(hi, zero) >= TOP_K, zero, i16_min)
    for b in range(14, -1, -1):
        cand = res | jnp.int32(1 << b)
        res = jnp.where(cnt_ge(hi, cand) >= TOP_K, cand, res)
    p = res
    pb = jnp.broadcast_to(p.astype(jnp.int16), hi.shape)
    cnt_above = fold_sum(jnp.where(hi > pb, one16, zero16))
    target = TOP_K - cnt_above
    lob = jnp.where(hi == pb, lo, jnp.int16(-(2**15)))
    resb = jnp.where(cnt_ge(lob, zero) >= target, zero, i16_min)
    for b in range(14, -1, -1):
        cand = resb | jnp.int32(1 << b)
        resb = jnp.where(cnt_ge(lob, cand) >= target, cand, resb)
    kth_key = jnp.left_shift(p, 16) | ((resb & 0xFFFF) ^ 0x8000)
    kth_i = kth_key ^ (jnp.right_shift(kth_key, 31) & jnp.int32(0x7FFFFFFF))
    return jax.lax.bitcast_convert_type(kth_i, jnp.float32)


def _fused_kernel(pq_ref, pkv_ref, q_ref, k_ref, o_ref):
    sims = (
        jax.lax.dot_general(
            pq_ref[...], pkv_ref[...], (((1,), (1,)), ((), ())),
            preferred_element_type=jnp.float32,
        )
        * SCALE
    )
    kth = _row_kth(sims)
    att = (
        jax.lax.dot_general(
            q_ref[...], k_ref[...], (((1,), (1,)), ((), ())),
            preferred_element_type=jnp.float32,
        )
        * SCALE
    )
    o_ref[...] = jnp.where(sims < kth, jnp.float32(0.0), att)


@jax.jit
def kernel(ae_q, ae_kv, pe_q, pe_kv, Wq, Wk):
    proj = pl.pallas_call(
        _proj_kernel,
        grid=(8,),
        in_specs=[
            pl.BlockSpec((512, D), lambda i: (i, 0)),
            pl.BlockSpec((512, D), lambda i: (i, 0)),
            pl.BlockSpec((D, D), lambda i: (0, 0)),
        ],
        out_specs=pl.BlockSpec((512, D), lambda i: (i, 0)),
        out_shape=jax.ShapeDtypeStruct((B, D), jnp.float32),
    )
    query = proj(ae_q, pe_q, Wq)
    key_mat = proj(ae_kv, pe_kv, Wk)

    R = 256
    out = pl.pallas_call(
        _fused_kernel,
        grid=(B // R,),
        in_specs=[
            pl.BlockSpec((R, D), lambda i: (i, 0)),
            pl.BlockSpec((KNOW, D), lambda i: (0, 0)),
            pl.BlockSpec((R, D), lambda i: (i, 0)),
            pl.BlockSpec((KNOW, D), lambda i: (0, 0)),
        ],
        out_specs=pl.BlockSpec((R, KNOW), lambda i: (i, 0)),
        out_shape=jax.ShapeDtypeStruct((B, KNOW), jnp.float32),
    )(pe_q, pe_kv, query, key_mat)
    return out
